# baseline (speedup 1.0000x reference)
import jax
import jax.numpy as jnp
from jax.experimental import pallas as pl

NS = 4096


def _copy_kernel(x_ref, y_ref, o1_ref, o2_ref):
    o1_ref[...] = x_ref[...]
    o2_ref[...] = y_ref[...]


def kernel(xyz, feature, raw_relative_feature, neighbors_idx):
    B, N = xyz.shape[0], xyz.shape[1]
    freqs = jax.vmap(lambda ni: jnp.zeros((N,), jnp.float32).at[ni.reshape(-1)].add(1.0))(neighbors_idx)
    weights = 1.0 / freqs
    g = jax.random.gumbel(jax.random.key(42), weights.shape, dtype=jnp.float32)
    scores = jnp.log(weights) + g
    indexes = jax.lax.top_k(scores, NS)[1]
    new_xyz = jnp.take_along_axis(xyz, indexes[:, :, None], axis=1)
    new_feature = jnp.take_along_axis(feature, indexes[:, :, None], axis=1)
    out = pl.pallas_call(
        _copy_kernel,
        grid=(B,),
        in_specs=[
            pl.BlockSpec((1, NS, 3), lambda b: (b, 0, 0)),
            pl.BlockSpec((1, NS, 128), lambda b: (b, 0, 0)),
        ],
        out_specs=[
            pl.BlockSpec((1, NS, 3), lambda b: (b, 0, 0)),
            pl.BlockSpec((1, NS, 128), lambda b: (b, 0, 0)),
        ],
        out_shape=(
            jax.ShapeDtypeStruct(new_xyz.shape, new_xyz.dtype),
            jax.ShapeDtypeStruct(new_feature.shape, new_feature.dtype),
        ),
    )(new_xyz, new_feature)
    return out


# speedup vs baseline: 2.9271x; 2.9271x over previous
import functools
import jax
import jax.numpy as jnp
from jax import lax
from jax.experimental import pallas as pl
from jax.experimental.pallas import tpu as pltpu, tpu_sc as plsc

B, N, K, C, NS = 8, 16384, 16, 128, 4096
MAXC = N * K
IDX_PER_TILE = (B * N * K) // 32
QN = N // 4

_mesh = plsc.VectorSubcoreMesh(core_axis_name="c", subcore_axis_name="s")
_sc_params = pltpu.CompilerParams(needs_layout_passes=False)


@functools.partial(
    pl.kernel,
    mesh=_mesh,
    out_type=jax.ShapeDtypeStruct((B * N,), jnp.float32),
    scratch_types=[
        pltpu.VMEM((IDX_PER_TILE,), jnp.int32),
        pltpu.VMEM((N,), jnp.int32),
        pltpu.VMEM((QN,), jnp.int32),
        pltpu.VMEM((QN,), jnp.int32),
        pltpu.VMEM((QN,), jnp.float32),
        pltpu.VMEM((QN,), jnp.float32),
        pltpu.VMEM_SHARED((16, N), jnp.int32),
        pltpu.SemaphoreType.DMA,
    ],
    compiler_params=_sc_params,
)
def _scores_sc(nid_hbm, lut_hbm, g_hbm, scores_out,
               idx_v, hist_v, acc_v, tmp_v, lutv_v, g_v, shared, sem):
    c = lax.axis_index("c")
    s = lax.axis_index("s")
    batch = c * 4 + s // 4
    quarter = s % 4
    nbase = batch * (N * K) + quarter * IDX_PER_TILE
    sbase = batch * N + quarter * QN

    pltpu.sync_copy(nid_hbm.at[pl.ds(nbase, IDX_PER_TILE)], idx_v)

    def zero_body(i, _):
        for j in range(8):
            hist_v[pl.ds(i * 128 + j * 16, 16)] = jnp.zeros((16,), jnp.int32)
        return 0
    lax.fori_loop(0, N // 128, zero_body, 0)

    ones = jnp.ones((16,), jnp.int32)

    def scat_body(i, _):
        for j in range(8):
            iv = idx_v[pl.ds(i * 128 + j * 16, 16)]
            plsc.addupdate_scatter(hist_v, [iv], ones)
        return 0
    lax.fori_loop(0, IDX_PER_TILE // 128, scat_body, 0)

    pltpu.sync_copy(hist_v, shared.at[s])
    plsc.subcore_barrier()

    group = s - quarter
    qoff = quarter * QN
    pltpu.sync_copy(shared.at[group, pl.ds(qoff, QN)], acc_v)
    for j in range(1, 4):
        pltpu.sync_copy(shared.at[group + j, pl.ds(qoff, QN)], tmp_v)

        def add_body(i, _):
            acc_v[pl.ds(i * 16, 16)] = acc_v[pl.ds(i * 16, 16)] + tmp_v[pl.ds(i * 16, 16)]
            return 0
        lax.fori_loop(0, QN // 16, add_body, 0)

    pltpu.async_copy(lut_hbm.at[acc_v], lutv_v, sem).wait()
    pltpu.sync_copy(g_hbm.at[pl.ds(sbase, QN)], g_v)

    def score_body(i, _):
        lutv_v[pl.ds(i * 16, 16)] = lutv_v[pl.ds(i * 16, 16)] + g_v[pl.ds(i * 16, 16)]
        return 0
    lax.fori_loop(0, QN // 16, score_body, 0)

    pltpu.sync_copy(lutv_v, scores_out.at[pl.ds(sbase, QN)])


def _sort_kernel(s_ref, o_ref):
    s = s_ref[...]
    pos = lax.broadcasted_iota(jnp.int32, (B, N), 1)
    si = pos
    for k_log in range(1, 15):
        k = 1 << k_log
        for j_log in range(k_log - 1, -1, -1):
            d = 1 << j_log
            is_lower = (pos & d) == 0
            bit_k = (pos & k) != 0
            ps = jnp.where(is_lower, jnp.roll(s, -d, axis=1), jnp.roll(s, d, axis=1))
            pi = jnp.where(is_lower, jnp.roll(si, -d, axis=1), jnp.roll(si, d, axis=1))
            g = (s > ps) | ((s == ps) & (si < pi))
            keep = g == (is_lower ^ bit_k)
            s = jnp.where(keep, s, ps)
            si = jnp.where(keep, si, pi)
    o_ref[...] = si[:, :NS]


def _sort_tc(scores):
    return pl.pallas_call(
        _sort_kernel,
        out_shape=jax.ShapeDtypeStruct((B, NS), jnp.int32),
    )(scores)


RPT = (B * NS) // 32
HALF = RPT // 2


@functools.partial(
    pl.kernel,
    mesh=_mesh,
    out_type=[
        jax.ShapeDtypeStruct((B * NS,), jnp.float32),
        jax.ShapeDtypeStruct((B * NS,), jnp.float32),
        jax.ShapeDtypeStruct((B * NS,), jnp.float32),
        jax.ShapeDtypeStruct((B * NS, C), jnp.float32),
    ],
    scratch_types=[
        pltpu.VMEM((HALF,), jnp.int32),
        pltpu.VMEM((HALF,), jnp.float32),
        pltpu.VMEM((HALF, C), jnp.float32),
        pltpu.SemaphoreType.DMA,
    ],
    compiler_params=_sc_params,
)
def _gather_sc(idxs_hbm, x_hbm, y_hbm, z_hbm, feat_hbm,
               x_out, y_out, z_out, feat_out,
               idx_v, pl_v, featr_v, sem):
    c = lax.axis_index("c")
    s = lax.axis_index("s")
    w = s * 2 + c
    rbase = w * RPT
    b = w // 4
    for h in range(2):
        hb = rbase + h * HALF
        pltpu.sync_copy(idxs_hbm.at[pl.ds(hb, HALF)], idx_v)

        def add_body(i, _):
            idx_v[pl.ds(i * 16, 16)] = idx_v[pl.ds(i * 16, 16)] + b * N
            return 0
        lax.fori_loop(0, HALF // 16, add_body, 0)

        for plane, plane_out in ((x_hbm, x_out), (y_hbm, y_out), (z_hbm, z_out)):
            pltpu.async_copy(plane.at[idx_v], pl_v, sem).wait()
            pltpu.sync_copy(pl_v, plane_out.at[pl.ds(hb, HALF)])
        pltpu.async_copy(feat_hbm.at[idx_v], featr_v, sem).wait()
        pltpu.sync_copy(featr_v, feat_out.at[pl.ds(hb, HALF)])


def kernel(xyz, feature, raw_relative_feature, neighbors_idx):
    g = jax.random.gumbel(jax.random.key(42), (B, N), dtype=jnp.float32)
    lut = jnp.log(1.0 / jnp.arange(MAXC + 1, dtype=jnp.float32))
    scores = _scores_sc(neighbors_idx.reshape(-1), lut, g.reshape(-1))
    idxs = _sort_tc(scores.reshape(B, N))
    xyz_flat = xyz.reshape(B * N, 3)
    xo, yo, zo, new_feature = _gather_sc(
        idxs.reshape(-1),
        xyz_flat[:, 0], xyz_flat[:, 1], xyz_flat[:, 2],
        feature.reshape(B * N, C),
    )
    new_xyz = jnp.stack([xo, yo, zo], axis=-1).reshape(B, NS, 3)
    return new_xyz, new_feature.reshape(B, NS, C)


# speedup vs baseline: 2.9828x; 1.0190x over previous
import functools
import jax
import jax.numpy as jnp
from jax import lax
from jax.experimental import pallas as pl
from jax.experimental.pallas import tpu as pltpu, tpu_sc as plsc

B, N, K, C, NS = 8, 16384, 16, 128, 4096
MAXC = N * K
IDX_PER_TILE = (B * N * K) // 32
QN = N // 4

_mesh = plsc.VectorSubcoreMesh(core_axis_name="c", subcore_axis_name="s")
_sc_params = pltpu.CompilerParams(needs_layout_passes=False)


U = 4
CHUNK = IDX_PER_TILE // 2


@functools.partial(
    pl.kernel,
    mesh=_mesh,
    out_type=jax.ShapeDtypeStruct((B * N,), jnp.float32),
    scratch_types=[
        pltpu.VMEM((CHUNK,), jnp.int32),
        pltpu.VMEM((U * N,), jnp.int32),
        pltpu.VMEM((QN,), jnp.int32),
        pltpu.VMEM((QN,), jnp.int32),
        pltpu.VMEM((QN,), jnp.float32),
        pltpu.VMEM((QN,), jnp.float32),
        pltpu.VMEM_SHARED((16, N), jnp.int32),
        pltpu.SemaphoreType.DMA,
    ],
    compiler_params=_sc_params,
)
def _scores_sc(nid_hbm, lut_hbm, g_hbm, scores_out,
               idx_v, hist_v, acc_v, tmp_v, lutv_v, g_v, shared, sem):
    c = lax.axis_index("c")
    s = lax.axis_index("s")
    batch = c * 4 + s // 4
    quarter = s % 4
    nbase = batch * (N * K) + quarter * IDX_PER_TILE
    sbase = batch * N + quarter * QN

    @plsc.parallel_loop(0, U * N // 128, unroll=2)
    def _zero(i):
        for j in range(8):
            hist_v[pl.ds(i * 128 + j * 16, 16)] = jnp.zeros((16,), jnp.int32)

    ones = jnp.ones((16,), jnp.int32)
    for ch in range(IDX_PER_TILE // CHUNK):
        pltpu.sync_copy(nid_hbm.at[pl.ds(nbase + ch * CHUNK, CHUNK)], idx_v)

        @plsc.parallel_loop(0, CHUNK // (U * 16), unroll=2)
        def _scat(i):
            for j in range(U):
                iv = idx_v[pl.ds(i * (U * 16) + j * 16, 16)]
                plsc.addupdate_scatter(hist_v, [iv + j * N], ones)

    @plsc.parallel_loop(0, N // 128, unroll=2)
    def _red(i):
        for j in range(8):
            off = i * 128 + j * 16
            acc = hist_v[pl.ds(off, 16)]
            for u in range(1, U):
                acc = acc + hist_v[pl.ds(u * N + off, 16)]
            hist_v[pl.ds(off, 16)] = acc

    pltpu.sync_copy(hist_v.at[pl.ds(0, N)], shared.at[s])
    plsc.subcore_barrier()

    group = s - quarter
    qoff = quarter * QN
    pltpu.sync_copy(shared.at[group, pl.ds(qoff, QN)], acc_v)
    for j in range(1, 4):
        pltpu.sync_copy(shared.at[group + j, pl.ds(qoff, QN)], tmp_v)

        @plsc.parallel_loop(0, QN // 128, unroll=2)
        def _add(i):
            for u in range(8):
                off = i * 128 + u * 16
                acc_v[pl.ds(off, 16)] = acc_v[pl.ds(off, 16)] + tmp_v[pl.ds(off, 16)]

    pltpu.async_copy(lut_hbm.at[acc_v], lutv_v, sem).wait()
    pltpu.sync_copy(g_hbm.at[pl.ds(sbase, QN)], g_v)

    @plsc.parallel_loop(0, QN // 128, unroll=2)
    def _score(i):
        for u in range(8):
            off = i * 128 + u * 16
            lutv_v[pl.ds(off, 16)] = lutv_v[pl.ds(off, 16)] + g_v[pl.ds(off, 16)]

    pltpu.sync_copy(lutv_v, scores_out.at[pl.ds(sbase, QN)])


def _sort_kernel(s_ref, o_ref):
    s = s_ref[...]
    pos = lax.broadcasted_iota(jnp.int32, (B, N), 1)
    si = pos
    for k_log in range(1, 15):
        k = 1 << k_log
        for j_log in range(k_log - 1, -1, -1):
            d = 1 << j_log
            is_lower = (pos & d) == 0
            bit_k = (pos & k) != 0
            ps = jnp.where(is_lower, jnp.roll(s, -d, axis=1), jnp.roll(s, d, axis=1))
            pi = jnp.where(is_lower, jnp.roll(si, -d, axis=1), jnp.roll(si, d, axis=1))
            g = (s > ps) | ((s == ps) & (si < pi))
            keep = g == (is_lower ^ bit_k)
            s = jnp.where(keep, s, ps)
            si = jnp.where(keep, si, pi)
    o_ref[...] = si[:, :NS]


def _sort_tc(scores):
    return pl.pallas_call(
        _sort_kernel,
        out_shape=jax.ShapeDtypeStruct((B, NS), jnp.int32),
    )(scores)


RPT = (B * NS) // 32
HALF = RPT // 2


@functools.partial(
    pl.kernel,
    mesh=_mesh,
    out_type=[
        jax.ShapeDtypeStruct((B * NS,), jnp.float32),
        jax.ShapeDtypeStruct((B * NS,), jnp.float32),
        jax.ShapeDtypeStruct((B * NS,), jnp.float32),
        jax.ShapeDtypeStruct((B * NS, C), jnp.float32),
    ],
    scratch_types=[
        pltpu.VMEM((HALF,), jnp.int32),
        pltpu.VMEM((HALF,), jnp.float32),
        pltpu.VMEM((HALF, C), jnp.float32),
        pltpu.SemaphoreType.DMA,
    ],
    compiler_params=_sc_params,
)
def _gather_sc(idxs_hbm, x_hbm, y_hbm, z_hbm, feat_hbm,
               x_out, y_out, z_out, feat_out,
               idx_v, pl_v, featr_v, sem):
    c = lax.axis_index("c")
    s = lax.axis_index("s")
    w = s * 2 + c
    rbase = w * RPT
    b = w // 4
    for h in range(2):
        hb = rbase + h * HALF
        pltpu.sync_copy(idxs_hbm.at[pl.ds(hb, HALF)], idx_v)

        def add_body(i, _):
            idx_v[pl.ds(i * 16, 16)] = idx_v[pl.ds(i * 16, 16)] + b * N
            return 0
        lax.fori_loop(0, HALF // 16, add_body, 0)

        for plane, plane_out in ((x_hbm, x_out), (y_hbm, y_out), (z_hbm, z_out)):
            pltpu.async_copy(plane.at[idx_v], pl_v, sem).wait()
            pltpu.sync_copy(pl_v, plane_out.at[pl.ds(hb, HALF)])
        pltpu.async_copy(feat_hbm.at[idx_v], featr_v, sem).wait()
        pltpu.sync_copy(featr_v, feat_out.at[pl.ds(hb, HALF)])


def kernel(xyz, feature, raw_relative_feature, neighbors_idx):
    g = jax.random.gumbel(jax.random.key(42), (B, N), dtype=jnp.float32)
    lut = jnp.log(1.0 / jnp.arange(MAXC + 1, dtype=jnp.float32))
    scores = _scores_sc(neighbors_idx.reshape(-1), lut, g.reshape(-1))
    idxs = _sort_tc(scores.reshape(B, N))
    xyz_flat = xyz.reshape(B * N, 3)
    xo, yo, zo, new_feature = _gather_sc(
        idxs.reshape(-1),
        xyz_flat[:, 0], xyz_flat[:, 1], xyz_flat[:, 2],
        feature.reshape(B * N, C),
    )
    new_xyz = jnp.stack([xo, yo, zo], axis=-1).reshape(B, NS, 3)
    return new_xyz, new_feature.reshape(B, NS, C)


# speedup vs baseline: 12.5863x; 4.2196x over previous
import functools
import jax
import jax.numpy as jnp
from jax import lax
from jax.experimental import pallas as pl
from jax.experimental.pallas import tpu as pltpu, tpu_sc as plsc

B, N, K, C, NS = 8, 16384, 16, 128, 4096
MAXC = N * K
IDX_PER_TILE = (B * N * K) // 32
QN = N // 4

_mesh = plsc.VectorSubcoreMesh(core_axis_name="c", subcore_axis_name="s")
_sc_params = pltpu.CompilerParams(needs_layout_passes=False)


U = 2
CHUNK = IDX_PER_TILE // 2
LUTS = 4096


@functools.partial(
    pl.kernel,
    mesh=_mesh,
    out_type=jax.ShapeDtypeStruct((B * N,), jnp.float32),
    scratch_types=[
        pltpu.VMEM((CHUNK,), jnp.int32),
        pltpu.VMEM((U * N,), jnp.int32),
        pltpu.VMEM((QN,), jnp.int32),
        pltpu.VMEM((QN,), jnp.int32),
        pltpu.VMEM((QN,), jnp.float32),
        pltpu.VMEM((QN,), jnp.float32),
        pltpu.VMEM((LUTS,), jnp.float32),
        pltpu.VMEM_SHARED((16, N), jnp.int32),
        pltpu.SemaphoreType.DMA,
    ],
    compiler_params=_sc_params,
)
def _scores_sc(nid_hbm, lut_hbm, g_hbm, scores_out,
               idx_v, hist_v, acc_v, tmp_v, lutv_v, g_v, lutsm_v, shared, sem):
    c = lax.axis_index("c")
    s = lax.axis_index("s")
    batch = c * 4 + s // 4
    quarter = s % 4
    nbase = batch * (N * K) + quarter * IDX_PER_TILE
    sbase = batch * N + quarter * QN

    @plsc.parallel_loop(0, U * N // 128, unroll=2)
    def _zero(i):
        for j in range(8):
            hist_v[pl.ds(i * 128 + j * 16, 16)] = jnp.zeros((16,), jnp.int32)

    ones = jnp.ones((16,), jnp.int32)
    for ch in range(IDX_PER_TILE // CHUNK):
        pltpu.sync_copy(nid_hbm.at[pl.ds(nbase + ch * CHUNK, CHUNK)], idx_v)

        @plsc.parallel_loop(0, CHUNK // (U * 16), unroll=2)
        def _scat(i):
            for j in range(U):
                iv = idx_v[pl.ds(i * (U * 16) + j * 16, 16)]
                plsc.addupdate_scatter(hist_v, [iv + j * N], ones)

    @plsc.parallel_loop(0, N // 128, unroll=2)
    def _red(i):
        for j in range(8):
            off = i * 128 + j * 16
            acc = hist_v[pl.ds(off, 16)]
            for u in range(1, U):
                acc = acc + hist_v[pl.ds(u * N + off, 16)]
            hist_v[pl.ds(off, 16)] = acc

    pltpu.sync_copy(hist_v.at[pl.ds(0, N)], shared.at[s])
    pltpu.sync_copy(lut_hbm.at[pl.ds(0, LUTS)], lutsm_v)
    plsc.subcore_barrier()

    group = s - quarter
    qoff = quarter * QN
    pltpu.sync_copy(shared.at[group, pl.ds(qoff, QN)], acc_v)
    for j in range(1, 4):
        pltpu.sync_copy(shared.at[group + j, pl.ds(qoff, QN)], tmp_v)

        @plsc.parallel_loop(0, QN // 128, unroll=2)
        def _add(i):
            for u in range(8):
                off = i * 128 + u * 16
                acc_v[pl.ds(off, 16)] = acc_v[pl.ds(off, 16)] + tmp_v[pl.ds(off, 16)]

    @plsc.parallel_loop(0, QN // 16, unroll=4, carry=jnp.int32(0))
    def _lookup(i, m):
        av = acc_v[pl.ds(i * 16, 16)]
        clamped = jnp.minimum(av, LUTS - 1)
        lutv_v[pl.ds(i * 16, 16)] = plsc.load_gather(lutsm_v, [clamped])
        return jnp.maximum(m, lax.reduce_max(av, (0,)))

    @pl.when(_lookup > LUTS - 1)
    def _slow():
        pltpu.async_copy(lut_hbm.at[acc_v], lutv_v, sem).wait()

    pltpu.sync_copy(g_hbm.at[pl.ds(sbase, QN)], g_v)

    @plsc.parallel_loop(0, QN // 128, unroll=2)
    def _score(i):
        for u in range(8):
            off = i * 128 + u * 16
            lutv_v[pl.ds(off, 16)] = lutv_v[pl.ds(off, 16)] + g_v[pl.ds(off, 16)]

    pltpu.sync_copy(lutv_v, scores_out.at[pl.ds(sbase, QN)])


def _sort_kernel(s_ref, o_ref):
    s = s_ref[...]
    pos = lax.broadcasted_iota(jnp.int32, (B, N), 1)
    si = pos
    for k_log in range(1, 15):
        k = 1 << k_log
        for j_log in range(k_log - 1, -1, -1):
            d = 1 << j_log
            is_lower = (pos & d) == 0
            bit_k = (pos & k) != 0
            ps = jnp.where(is_lower, jnp.roll(s, -d, axis=1), jnp.roll(s, d, axis=1))
            pi = jnp.where(is_lower, jnp.roll(si, -d, axis=1), jnp.roll(si, d, axis=1))
            g = (s > ps) | ((s == ps) & (si < pi))
            keep = g == (is_lower ^ bit_k)
            s = jnp.where(keep, s, ps)
            si = jnp.where(keep, si, pi)
    o_ref[...] = si[:, :NS]


def _sort_tc(scores):
    return pl.pallas_call(
        _sort_kernel,
        out_shape=jax.ShapeDtypeStruct((B, NS), jnp.int32),
    )(scores)


RPT = (B * NS) // 32
HALF = RPT // 2


@functools.partial(
    pl.kernel,
    mesh=_mesh,
    out_type=[
        jax.ShapeDtypeStruct((B * NS,), jnp.float32),
        jax.ShapeDtypeStruct((B * NS,), jnp.float32),
        jax.ShapeDtypeStruct((B * NS,), jnp.float32),
        jax.ShapeDtypeStruct((B * NS, C), jnp.float32),
    ],
    scratch_types=[
        pltpu.VMEM((HALF,), jnp.int32),
        pltpu.VMEM((HALF,), jnp.float32),
        pltpu.VMEM((HALF, C), jnp.float32),
        pltpu.SemaphoreType.DMA,
    ],
    compiler_params=_sc_params,
)
def _gather_sc(idxs_hbm, x_hbm, y_hbm, z_hbm, feat_hbm,
               x_out, y_out, z_out, feat_out,
               idx_v, pl_v, featr_v, sem):
    c = lax.axis_index("c")
    s = lax.axis_index("s")
    w = s * 2 + c
    rbase = w * RPT
    b = w // 4
    for h in range(2):
        hb = rbase + h * HALF
        pltpu.sync_copy(idxs_hbm.at[pl.ds(hb, HALF)], idx_v)

        def add_body(i, _):
            idx_v[pl.ds(i * 16, 16)] = idx_v[pl.ds(i * 16, 16)] + b * N
            return 0
        lax.fori_loop(0, HALF // 16, add_body, 0)

        for plane, plane_out in ((x_hbm, x_out), (y_hbm, y_out), (z_hbm, z_out)):
            pltpu.async_copy(plane.at[idx_v], pl_v, sem).wait()
            pltpu.sync_copy(pl_v, plane_out.at[pl.ds(hb, HALF)])
        pltpu.async_copy(feat_hbm.at[idx_v], featr_v, sem).wait()
        pltpu.sync_copy(featr_v, feat_out.at[pl.ds(hb, HALF)])


def kernel(xyz, feature, raw_relative_feature, neighbors_idx):
    g = jax.random.gumbel(jax.random.key(42), (B, N), dtype=jnp.float32)
    lut = jnp.log(1.0 / jnp.arange(MAXC + 1, dtype=jnp.float32))
    scores = _scores_sc(neighbors_idx.reshape(-1), lut, g.reshape(-1))
    idxs = _sort_tc(scores.reshape(B, N))
    xyz_flat = xyz.reshape(B * N, 3)
    xo, yo, zo, new_feature = _gather_sc(
        idxs.reshape(-1),
        xyz_flat[:, 0], xyz_flat[:, 1], xyz_flat[:, 2],
        feature.reshape(B * N, C),
    )
    new_xyz = jnp.stack([xo, yo, zo], axis=-1).reshape(B, NS, 3)
    return new_xyz, new_feature.reshape(B, NS, C)
